# Initial kernel scaffold; baseline (speedup 1.0000x reference)
import jax
import jax.numpy as jnp
from jax.experimental import pallas as pl


def kernel(x, weight):
    raise NotImplementedError("write your pallas kernel here")


# speedup vs baseline: 1.5491x; 1.5491x over previous
import functools

import jax
import jax.numpy as jnp
from jax import lax
from jax.experimental import pallas as pl
from jax.experimental.pallas import tpu as pltpu
from jax.experimental.pallas import tpu_sc as plsc

DIM = 32
CHUNK = 128
K = 8


def _make_gather(n_total, nw):
  n_chunks = n_total // CHUNK
  ch_per_w = n_chunks // nw
  n_super = ch_per_w // K
  mesh = plsc.VectorSubcoreMesh(core_axis_name="c", subcore_axis_name="s")

  @functools.partial(
      pl.kernel,
      out_type=jax.ShapeDtypeStruct((n_chunks, CHUNK, DIM), jnp.float32),
      mesh=mesh,
      scratch_types=[
          pltpu.VMEM((K, CHUNK), jnp.int32),
          pltpu.VMEM((K, CHUNK, DIM), jnp.float32),
          pltpu.SemaphoreType.DMA,
      ],
      compiler_params=pltpu.CompilerParams(use_tc_tiling_on_sc=False),
  )
  def gather_kernel(idx_hbm, table_hbm, out_hbm, idx_v, rows_v, gsem):
    wid = lax.axis_index("s") * 2 + lax.axis_index("c")
    base = wid * ch_per_w

    def step(g, carry):
      row0 = base + g * K
      pltpu.sync_copy(idx_hbm.at[pl.ds(row0, K)], idx_v)
      copies = [
          pltpu.async_copy(table_hbm.at[idx_v.at[j]], rows_v.at[j], gsem)
          for j in range(K)
      ]
      for c in copies:
        c.wait()
      pltpu.sync_copy(rows_v, out_hbm.at[pl.ds(row0, K)])
      return carry

    lax.fori_loop(0, n_super, step, 0)

  return gather_kernel


def kernel(x, weight):
  b, f = x.shape
  n_total = b * f
  idx = x.reshape(n_total // CHUNK, CHUNK)
  out = _make_gather(n_total, 32)(idx, weight)
  return out.reshape(b, f, DIM)


# speedup vs baseline: 1.5708x; 1.0141x over previous
import functools

import jax
import jax.numpy as jnp
from jax import lax
from jax.experimental import pallas as pl
from jax.experimental.pallas import tpu as pltpu
from jax.experimental.pallas import tpu_sc as plsc

DIM = 32
CHUNK = 128
K = 13
NSUPER = 8
NW = 32


def _make_gather(n_total):
  n_chunks = n_total // CHUNK
  ch_per_w = n_chunks // NW
  mesh = plsc.VectorSubcoreMesh(core_axis_name="c", subcore_axis_name="s")

  @functools.partial(
      pl.kernel,
      out_type=jax.ShapeDtypeStruct((n_chunks, CHUNK, DIM), jnp.float32),
      mesh=mesh,
      scratch_types=[
          pltpu.VMEM((ch_per_w, CHUNK), jnp.int32),
          pltpu.VMEM((K, CHUNK, DIM), jnp.float32),
          pltpu.VMEM((K, CHUNK, DIM), jnp.float32),
          pltpu.SemaphoreType.DMA,
          pltpu.SemaphoreType.DMA,
          pltpu.SemaphoreType.DMA,
          pltpu.SemaphoreType.DMA,
      ],
      compiler_params=pltpu.CompilerParams(use_tc_tiling_on_sc=False),
  )
  def gather_kernel(idx_hbm, table_hbm, out_hbm, idx_v, rows_a, rows_b,
                    gsem_a, gsem_b, osem_a, osem_b):
    wid = lax.axis_index("s") * 2 + lax.axis_index("c")
    base = wid * ch_per_w
    pltpu.sync_copy(idx_hbm.at[pl.ds(base, ch_per_w)], idx_v)

    bufs = (rows_a, rows_b)
    gsems = (gsem_a, gsem_b)
    osems = (osem_a, osem_b)

    def fire(g):
      buf, gsem = bufs[g % 2], gsems[g % 2]
      return [
          pltpu.async_copy(table_hbm.at[idx_v.at[g * K + j]], buf.at[j], gsem)
          for j in range(K)
      ]

    out_copies = [None, None]
    gather_copies = [fire(0), None]
    for g in range(NSUPER):
      p = g % 2
      if g + 1 < NSUPER:
        if out_copies[1 - p] is not None:
          out_copies[1 - p].wait()
        gather_copies[1 - p] = fire(g + 1)
      for c in gather_copies[p]:
        c.wait()
      out_copies[p] = pltpu.async_copy(
          bufs[p], out_hbm.at[pl.ds(base + g * K, K)], osems[p])
    out_copies[0].wait()
    out_copies[1].wait()

  return gather_kernel


def kernel(x, weight):
  b, f = x.shape
  n_total = b * f
  idx = x.reshape(n_total // CHUNK, CHUNK)
  out = _make_gather(n_total)(idx, weight)
  return out.reshape(b, f, DIM)
